# baseline (device time: 39475 ns/iter reference)
import jax
import jax.numpy as jnp
from jax import lax
from jax.experimental import pallas as pl
from jax.experimental.pallas import tpu as pltpu

N_DEV = 8
B = 2
SQ = 512
SKV = 512
DH = 64
H_PER = 8
D_MODEL = 768
M = B * SQ
RC = SQ // N_DEV


def _body(x_ref, wq_ref, k_ref, v_ref, wo_ref, out_ref,
          pstage_ref, ctx_ref, rs_buf, ag_buf, red_ref,
          rs_send, rs_recv, ag_send, ag_recv):
    my = lax.axis_index("i")

    barrier_sem = pltpu.get_barrier_semaphore()
    for d in range(1, N_DEV):
        peer = lax.rem(my + d, N_DEV)
        pl.semaphore_signal(
            barrier_sem, inc=1,
            device_id=(peer,), device_id_type=pl.DeviceIdType.MESH,
        )
    pl.semaphore_wait(barrier_sem, N_DEV - 1)

    q_all = jnp.dot(
        x_ref[...], wq_ref[...], preferred_element_type=jnp.float32
    ) * 0.125
    q_all = q_all.astype(jnp.bfloat16)

    qi = lax.broadcasted_iota(jnp.int32, (SQ, SKV), 0)
    ki = lax.broadcasted_iota(jnp.int32, (SQ, SKV), 1)
    mask = (jnp.abs(qi - ki) <= 128) | (ki < 32) | (qi < 32)

    rs_sends = []
    ag_sends = []

    def rs_wave(b):
        for c in range(N_DEV):
            rdma = pltpu.make_async_remote_copy(
                src_ref=pstage_ref.at[pl.ds(b * SQ + c * RC, RC)],
                dst_ref=rs_buf.at[b, my],
                send_sem=rs_send.at[b * N_DEV + c],
                recv_sem=rs_recv.at[b * N_DEV + my],
                device_id=(c,),
                device_id_type=pl.DeviceIdType.MESH,
            )
            rdma.start()
            rs_sends.append(rdma)

    def reduce_and_ag_wave(b):
        acc = jnp.zeros((RC, D_MODEL), jnp.float32)
        for src in range(N_DEV):
            recv = pltpu.make_async_remote_copy(
                src_ref=pstage_ref.at[pl.ds(0, RC)],
                dst_ref=rs_buf.at[b, src],
                send_sem=rs_send.at[src],
                recv_sem=rs_recv.at[b * N_DEV + src],
                device_id=(src,),
                device_id_type=pl.DeviceIdType.MESH,
            )
            recv.wait_recv()
            acc = acc + rs_buf[b, src, :, :].astype(jnp.float32)
        red_ref[b, :, :] = acc.astype(jnp.bfloat16)
        for d in range(N_DEV):
            peer = lax.rem(my + d, N_DEV)
            rdma = pltpu.make_async_remote_copy(
                src_ref=red_ref.at[b],
                dst_ref=ag_buf.at[b, my],
                send_sem=ag_send.at[b * N_DEV + d],
                recv_sem=ag_recv.at[b * N_DEV + my],
                device_id=(peer,),
                device_id_type=pl.DeviceIdType.MESH,
            )
            rdma.start()
            ag_sends.append(rdma)

    for b in range(B):
        for h in range(H_PER):
            bh = b * H_PER + h
            q_h = q_all[b * SQ:(b + 1) * SQ, h * DH:(h + 1) * DH]
            s = lax.dot_general(
                q_h, k_ref[bh, :, :],
                (((1,), (1,)), ((), ())),
                preferred_element_type=jnp.float32,
            )
            e = jnp.where(mask, jnp.exp(s), 0.0)
            denom = jnp.sum(e, axis=1, keepdims=True)
            ctx_h = jnp.dot(
                e.astype(jnp.bfloat16), v_ref[bh, :, :],
                preferred_element_type=jnp.float32,
            ) * (1.0 / denom)
            ctx_ref[:, h * DH:(h + 1) * DH] = ctx_h.astype(jnp.bfloat16)

        part_b = jnp.dot(
            ctx_ref[...], wo_ref[...], preferred_element_type=jnp.float32
        )
        pstage_ref[pl.ds(b * SQ, SQ), :] = part_b.astype(jnp.bfloat16)
        rs_wave(b)
        if b > 0:
            reduce_and_ag_wave(b - 1)
    reduce_and_ag_wave(B - 1)

    for rdma in rs_sends:
        rdma.wait_send()

    for b in range(B):
        for src in range(N_DEV):
            recv = pltpu.make_async_remote_copy(
                src_ref=red_ref.at[0],
                dst_ref=ag_buf.at[b, src],
                send_sem=ag_send.at[src],
                recv_sem=ag_recv.at[b * N_DEV + src],
                device_id=(src,),
                device_id_type=pl.DeviceIdType.MESH,
            )
            recv.wait_recv()
            out_ref[pl.ds(b * SQ + src * RC, RC), :] = ag_buf[
                b, src, :, :
            ].astype(jnp.float32)

    for rdma in ag_sends:
        rdma.wait_send()


def kernel(x, Wq, K_ext, V_ext, Wo):
    my = lax.axis_index("i")

    xb = x.reshape(M, D_MODEL).astype(jnp.bfloat16)
    wq = Wq.astype(jnp.bfloat16)
    wo = Wo.astype(jnp.bfloat16)
    k = lax.dynamic_slice_in_dim(K_ext, my * H_PER, H_PER, axis=2)
    v = lax.dynamic_slice_in_dim(V_ext, my * H_PER, H_PER, axis=2)
    k = k.transpose(0, 2, 1, 3).reshape(B * H_PER, SKV, DH).astype(jnp.bfloat16)
    v = v.transpose(0, 2, 1, 3).reshape(B * H_PER, SKV, DH).astype(jnp.bfloat16)

    out = pl.pallas_call(
        _body,
        out_shape=jax.ShapeDtypeStruct((M, D_MODEL), jnp.float32),
        in_specs=[pl.BlockSpec(memory_space=pltpu.VMEM)] * 5,
        out_specs=pl.BlockSpec(memory_space=pltpu.VMEM),
        scratch_shapes=[
            pltpu.VMEM((M, D_MODEL), jnp.bfloat16),
            pltpu.VMEM((SQ, H_PER * DH), jnp.bfloat16),
            pltpu.VMEM((B, N_DEV, RC, D_MODEL), jnp.bfloat16),
            pltpu.VMEM((B, N_DEV, RC, D_MODEL), jnp.bfloat16),
            pltpu.VMEM((B, RC, D_MODEL), jnp.bfloat16),
            pltpu.SemaphoreType.DMA((B * N_DEV,)),
            pltpu.SemaphoreType.DMA((B * N_DEV,)),
            pltpu.SemaphoreType.DMA((B * N_DEV,)),
            pltpu.SemaphoreType.DMA((B * N_DEV,)),
        ],
        compiler_params=pltpu.CompilerParams(collective_id=0),
    )(xb, wq, k, v, wo)
    return out.reshape(B, SQ, D_MODEL)


# device time: 34780 ns/iter; 1.1350x vs baseline; 1.1350x over previous
import jax
import jax.numpy as jnp
from jax import lax
from jax.experimental import pallas as pl
from jax.experimental.pallas import tpu as pltpu

N_DEV = 8
B = 2
SQ = 512
SKV = 512
DH = 64
H_PER = 8
D_MODEL = 768
M = B * SQ
RC = SQ // N_DEV
SROWS = 8


def _quantize(vals):
    amax = jnp.max(jnp.abs(vals))
    scale = amax * (1.0 / 127.0) + 1e-30
    q = jnp.round(vals * (1.0 / scale)).astype(jnp.int8)
    return q, scale


def _body(x_ref, wq_ref, k_ref, v_ref, wo_ref, out_ref,
          qstage_ref, sstage_ref, ctx_ref, rs_q, rs_s, ag_q, ag_s,
          agq_stage, ags_stage,
          rs_send, rs_recv, ag_send, ag_recv):
    my = lax.axis_index("i")

    barrier_sem = pltpu.get_barrier_semaphore()
    for d in range(1, N_DEV):
        peer = lax.rem(my + d, N_DEV)
        pl.semaphore_signal(
            barrier_sem, inc=1,
            device_id=(peer,), device_id_type=pl.DeviceIdType.MESH,
        )
    pl.semaphore_wait(barrier_sem, N_DEV - 1)

    q_all = jnp.dot(
        x_ref[...], wq_ref[...], preferred_element_type=jnp.float32
    ) * 0.125
    q_all = q_all.astype(jnp.bfloat16)

    qi = lax.broadcasted_iota(jnp.int32, (SQ, SKV), 0)
    ki = lax.broadcasted_iota(jnp.int32, (SQ, SKV), 1)
    mask = (jnp.abs(qi - ki) <= 128) | (ki < 32) | (qi < 32)

    rs_sends = []
    ag_sends = []

    def rs_wave(b):
        for c in range(N_DEV):
            data = pltpu.make_async_remote_copy(
                src_ref=qstage_ref.at[pl.ds(b * SQ + c * RC, RC)],
                dst_ref=rs_q.at[b, my],
                send_sem=rs_send.at[2 * (b * N_DEV + c)],
                recv_sem=rs_recv.at[2 * (b * N_DEV + my)],
                device_id=(c,),
                device_id_type=pl.DeviceIdType.MESH,
            )
            data.start()
            rs_sends.append(data)
            sc = pltpu.make_async_remote_copy(
                src_ref=sstage_ref.at[b],
                dst_ref=rs_s.at[b, my],
                send_sem=rs_send.at[2 * (b * N_DEV + c) + 1],
                recv_sem=rs_recv.at[2 * (b * N_DEV + my) + 1],
                device_id=(c,),
                device_id_type=pl.DeviceIdType.MESH,
            )
            sc.start()
            rs_sends.append(sc)

    def reduce_and_ag_wave(b):
        acc = jnp.zeros((RC, D_MODEL), jnp.float32)
        for src in range(N_DEV):
            data = pltpu.make_async_remote_copy(
                src_ref=qstage_ref.at[pl.ds(0, RC)],
                dst_ref=rs_q.at[b, src],
                send_sem=rs_send.at[0],
                recv_sem=rs_recv.at[2 * (b * N_DEV + src)],
                device_id=(src,),
                device_id_type=pl.DeviceIdType.MESH,
            )
            data.wait_recv()
            sc = pltpu.make_async_remote_copy(
                src_ref=sstage_ref.at[0],
                dst_ref=rs_s.at[b, src],
                send_sem=rs_send.at[0],
                recv_sem=rs_recv.at[2 * (b * N_DEV + src) + 1],
                device_id=(src,),
                device_id_type=pl.DeviceIdType.MESH,
            )
            sc.wait_recv()
            acc = acc + (
                rs_q[b, src, :, :].astype(jnp.float32)
                * rs_s[b, src, 0:1, 0:1]
            )
        rq, rscale = _quantize(acc)
        agq_stage[b, :, :] = rq
        ags_stage[b, :, :] = jnp.full((SROWS, 128), rscale, jnp.float32)
        for d in range(N_DEV):
            peer = lax.rem(my + d, N_DEV)
            data = pltpu.make_async_remote_copy(
                src_ref=agq_stage.at[b],
                dst_ref=ag_q.at[b, my],
                send_sem=ag_send.at[2 * (b * N_DEV + d)],
                recv_sem=ag_recv.at[2 * (b * N_DEV + my)],
                device_id=(peer,),
                device_id_type=pl.DeviceIdType.MESH,
            )
            data.start()
            ag_sends.append(data)
            sc = pltpu.make_async_remote_copy(
                src_ref=ags_stage.at[b],
                dst_ref=ag_s.at[b, my],
                send_sem=ag_send.at[2 * (b * N_DEV + d) + 1],
                recv_sem=ag_recv.at[2 * (b * N_DEV + my) + 1],
                device_id=(peer,),
                device_id_type=pl.DeviceIdType.MESH,
            )
            sc.start()
            ag_sends.append(sc)

    for b in range(B):
        for h in range(H_PER):
            bh = b * H_PER + h
            q_h = q_all[b * SQ:(b + 1) * SQ, h * DH:(h + 1) * DH]
            s = lax.dot_general(
                q_h, k_ref[bh, :, :],
                (((1,), (1,)), ((), ())),
                preferred_element_type=jnp.float32,
            )
            e = jnp.where(mask, jnp.exp(s), 0.0)
            denom = jnp.sum(e, axis=1, keepdims=True)
            ctx_h = jnp.dot(
                e.astype(jnp.bfloat16), v_ref[bh, :, :],
                preferred_element_type=jnp.float32,
            ) * (1.0 / denom)
            ctx_ref[:, h * DH:(h + 1) * DH] = ctx_h.astype(jnp.bfloat16)

        part_b = jnp.dot(
            ctx_ref[...], wo_ref[...], preferred_element_type=jnp.float32
        )
        pq, pscale = _quantize(part_b)
        qstage_ref[pl.ds(b * SQ, SQ), :] = pq
        sstage_ref[b, :, :] = jnp.full((SROWS, 128), pscale, jnp.float32)
        rs_wave(b)
        if b > 0:
            reduce_and_ag_wave(b - 1)
    reduce_and_ag_wave(B - 1)

    for rdma in rs_sends:
        rdma.wait_send()

    for b in range(B):
        for src in range(N_DEV):
            data = pltpu.make_async_remote_copy(
                src_ref=agq_stage.at[0],
                dst_ref=ag_q.at[b, src],
                send_sem=ag_send.at[0],
                recv_sem=ag_recv.at[2 * (b * N_DEV + src)],
                device_id=(src,),
                device_id_type=pl.DeviceIdType.MESH,
            )
            data.wait_recv()
            sc = pltpu.make_async_remote_copy(
                src_ref=ags_stage.at[0],
                dst_ref=ag_s.at[b, src],
                send_sem=ag_send.at[0],
                recv_sem=ag_recv.at[2 * (b * N_DEV + src) + 1],
                device_id=(src,),
                device_id_type=pl.DeviceIdType.MESH,
            )
            sc.wait_recv()
            out_ref[pl.ds(b * SQ + src * RC, RC), :] = (
                ag_q[b, src, :, :].astype(jnp.float32)
                * ag_s[b, src, 0:1, 0:1]
            )

    for rdma in ag_sends:
        rdma.wait_send()


def kernel(x, Wq, K_ext, V_ext, Wo):
    my = lax.axis_index("i")

    xb = x.reshape(M, D_MODEL).astype(jnp.bfloat16)
    wq = Wq.astype(jnp.bfloat16)
    wo = Wo.astype(jnp.bfloat16)
    k = lax.dynamic_slice_in_dim(K_ext, my * H_PER, H_PER, axis=2)
    v = lax.dynamic_slice_in_dim(V_ext, my * H_PER, H_PER, axis=2)
    k = k.transpose(0, 2, 1, 3).reshape(B * H_PER, SKV, DH).astype(jnp.bfloat16)
    v = v.transpose(0, 2, 1, 3).reshape(B * H_PER, SKV, DH).astype(jnp.bfloat16)

    out = pl.pallas_call(
        _body,
        out_shape=jax.ShapeDtypeStruct((M, D_MODEL), jnp.float32),
        in_specs=[pl.BlockSpec(memory_space=pltpu.VMEM)] * 5,
        out_specs=pl.BlockSpec(memory_space=pltpu.VMEM),
        scratch_shapes=[
            pltpu.VMEM((M, D_MODEL), jnp.int8),
            pltpu.VMEM((B, SROWS, 128), jnp.float32),
            pltpu.VMEM((SQ, H_PER * DH), jnp.bfloat16),
            pltpu.VMEM((B, N_DEV, RC, D_MODEL), jnp.int8),
            pltpu.VMEM((B, N_DEV, SROWS, 128), jnp.float32),
            pltpu.VMEM((B, N_DEV, RC, D_MODEL), jnp.int8),
            pltpu.VMEM((B, N_DEV, SROWS, 128), jnp.float32),
            pltpu.VMEM((B, RC, D_MODEL), jnp.int8),
            pltpu.VMEM((B, SROWS, 128), jnp.float32),
            pltpu.SemaphoreType.DMA((2 * B * N_DEV,)),
            pltpu.SemaphoreType.DMA((2 * B * N_DEV,)),
            pltpu.SemaphoreType.DMA((2 * B * N_DEV,)),
            pltpu.SemaphoreType.DMA((2 * B * N_DEV,)),
        ],
        compiler_params=pltpu.CompilerParams(collective_id=0),
    )(xb, wq, k, v, wo)
    return out.reshape(B, SQ, D_MODEL)


# device time: 33901 ns/iter; 1.1644x vs baseline; 1.0259x over previous
import jax
import jax.numpy as jnp
from jax import lax
from jax.experimental import pallas as pl
from jax.experimental.pallas import tpu as pltpu

N_DEV = 8
B = 2
SQ = 512
SKV = 512
DH = 64
H_PER = 8
D_MODEL = 768
M = B * SQ
RC = SQ // N_DEV
SROWS = 8


def _quantize(vals):
    amax = jnp.max(jnp.abs(vals))
    scale = amax * (1.0 / 127.0) + 1e-30
    q = jnp.round(vals * (1.0 / scale)).astype(jnp.int8)
    return q, scale


def _body(x_ref, wq_ref, k_ref, v_ref, wo_ref, out_ref,
          qstage_ref, sstage_ref, ctx_ref, rs_q, rs_s, ag_q, ag_s,
          agq_stage, ags_stage,
          rs_send, rs_recv, ag_send, ag_recv):
    my = lax.axis_index("i")

    barrier_sem = pltpu.get_barrier_semaphore()
    for d in range(1, N_DEV):
        peer = lax.rem(my + d, N_DEV)
        pl.semaphore_signal(
            barrier_sem, inc=1,
            device_id=(peer,), device_id_type=pl.DeviceIdType.MESH,
        )
    pl.semaphore_wait(barrier_sem, N_DEV - 1)

    q_all = jnp.dot(
        x_ref[...].astype(jnp.bfloat16), wq_ref[...],
        preferred_element_type=jnp.float32,
    ) * 0.125
    q_all = q_all.astype(jnp.bfloat16)

    qi = lax.broadcasted_iota(jnp.int32, (SQ, SKV), 0)
    ki = lax.broadcasted_iota(jnp.int32, (SQ, SKV), 1)
    mask = (jnp.abs(qi - ki) <= 128) | (ki < 32) | (qi < 32)

    rs_sends = []
    ag_sends = []

    def rs_wave(b):
        for c in range(N_DEV):
            data = pltpu.make_async_remote_copy(
                src_ref=qstage_ref.at[pl.ds(b * SQ + c * RC, RC)],
                dst_ref=rs_q.at[b, my],
                send_sem=rs_send.at[2 * (b * N_DEV + c)],
                recv_sem=rs_recv.at[2 * (b * N_DEV + my)],
                device_id=(c,),
                device_id_type=pl.DeviceIdType.MESH,
            )
            data.start()
            rs_sends.append(data)
            sc = pltpu.make_async_remote_copy(
                src_ref=sstage_ref.at[b],
                dst_ref=rs_s.at[b, my],
                send_sem=rs_send.at[2 * (b * N_DEV + c) + 1],
                recv_sem=rs_recv.at[2 * (b * N_DEV + my) + 1],
                device_id=(c,),
                device_id_type=pl.DeviceIdType.MESH,
            )
            sc.start()
            rs_sends.append(sc)

    def reduce_and_ag_wave(b):
        acc = jnp.zeros((RC, D_MODEL), jnp.float32)
        for src in range(N_DEV):
            data = pltpu.make_async_remote_copy(
                src_ref=qstage_ref.at[pl.ds(0, RC)],
                dst_ref=rs_q.at[b, src],
                send_sem=rs_send.at[0],
                recv_sem=rs_recv.at[2 * (b * N_DEV + src)],
                device_id=(src,),
                device_id_type=pl.DeviceIdType.MESH,
            )
            data.wait_recv()
            sc = pltpu.make_async_remote_copy(
                src_ref=sstage_ref.at[0],
                dst_ref=rs_s.at[b, src],
                send_sem=rs_send.at[0],
                recv_sem=rs_recv.at[2 * (b * N_DEV + src) + 1],
                device_id=(src,),
                device_id_type=pl.DeviceIdType.MESH,
            )
            sc.wait_recv()
            acc = acc + (
                rs_q[b, src, :, :].astype(jnp.float32)
                * rs_s[b, src, 0:1, 0:1]
            )
        rq, rscale = _quantize(acc)
        agq_stage[b, :, :] = rq
        ags_stage[b, :, :] = jnp.full((SROWS, 128), rscale, jnp.float32)
        for d in range(N_DEV):
            peer = lax.rem(my + d, N_DEV)
            data = pltpu.make_async_remote_copy(
                src_ref=agq_stage.at[b],
                dst_ref=ag_q.at[b, my],
                send_sem=ag_send.at[2 * (b * N_DEV + d)],
                recv_sem=ag_recv.at[2 * (b * N_DEV + my)],
                device_id=(peer,),
                device_id_type=pl.DeviceIdType.MESH,
            )
            data.start()
            ag_sends.append(data)
            sc = pltpu.make_async_remote_copy(
                src_ref=ags_stage.at[b],
                dst_ref=ag_s.at[b, my],
                send_sem=ag_send.at[2 * (b * N_DEV + d) + 1],
                recv_sem=ag_recv.at[2 * (b * N_DEV + my) + 1],
                device_id=(peer,),
                device_id_type=pl.DeviceIdType.MESH,
            )
            sc.start()
            ag_sends.append(sc)

    for b in range(B):
        for h in range(H_PER):
            bh = b * H_PER + h
            q_h = q_all[b * SQ:(b + 1) * SQ, h * DH:(h + 1) * DH]
            s = lax.dot_general(
                q_h, k_ref[bh, :, :],
                (((1,), (1,)), ((), ())),
                preferred_element_type=jnp.float32,
            )
            e = jnp.where(mask, jnp.exp(s), 0.0)
            denom = jnp.sum(e, axis=1, keepdims=True)
            ctx_h = jnp.dot(
                e.astype(jnp.bfloat16), v_ref[bh, :, :],
                preferred_element_type=jnp.float32,
            ) * (1.0 / denom)
            ctx_ref[:, h * DH:(h + 1) * DH] = ctx_h.astype(jnp.bfloat16)

        part_b = jnp.dot(
            ctx_ref[...], wo_ref[...], preferred_element_type=jnp.float32
        )
        pq, pscale = _quantize(part_b)
        qstage_ref[pl.ds(b * SQ, SQ), :] = pq
        sstage_ref[b, :, :] = jnp.full((SROWS, 128), pscale, jnp.float32)
        rs_wave(b)
        if b > 0:
            reduce_and_ag_wave(b - 1)
    reduce_and_ag_wave(B - 1)

    for rdma in rs_sends:
        rdma.wait_send()

    for b in range(B):
        for src in range(N_DEV):
            data = pltpu.make_async_remote_copy(
                src_ref=agq_stage.at[0],
                dst_ref=ag_q.at[b, src],
                send_sem=ag_send.at[0],
                recv_sem=ag_recv.at[2 * (b * N_DEV + src)],
                device_id=(src,),
                device_id_type=pl.DeviceIdType.MESH,
            )
            data.wait_recv()
            sc = pltpu.make_async_remote_copy(
                src_ref=ags_stage.at[0],
                dst_ref=ag_s.at[b, src],
                send_sem=ag_send.at[0],
                recv_sem=ag_recv.at[2 * (b * N_DEV + src) + 1],
                device_id=(src,),
                device_id_type=pl.DeviceIdType.MESH,
            )
            sc.wait_recv()
            out_ref[pl.ds(b * SQ + src * RC, RC), :] = (
                ag_q[b, src, :, :].astype(jnp.float32)
                * ag_s[b, src, 0:1, 0:1]
            )

    for rdma in ag_sends:
        rdma.wait_send()


def kernel(x, Wq, K_ext, V_ext, Wo):
    my = lax.axis_index("i")

    xb = x.reshape(M, D_MODEL)
    wq = Wq.astype(jnp.bfloat16)
    wo = Wo.astype(jnp.bfloat16)
    k = lax.dynamic_slice_in_dim(K_ext, my * H_PER, H_PER, axis=2)
    v = lax.dynamic_slice_in_dim(V_ext, my * H_PER, H_PER, axis=2)
    k = k.astype(jnp.bfloat16).transpose(0, 2, 1, 3).reshape(B * H_PER, SKV, DH)
    v = v.astype(jnp.bfloat16).transpose(0, 2, 1, 3).reshape(B * H_PER, SKV, DH)

    out = pl.pallas_call(
        _body,
        out_shape=jax.ShapeDtypeStruct((M, D_MODEL), jnp.float32),
        in_specs=[pl.BlockSpec(memory_space=pltpu.VMEM)] * 5,
        out_specs=pl.BlockSpec(memory_space=pltpu.VMEM),
        scratch_shapes=[
            pltpu.VMEM((M, D_MODEL), jnp.int8),
            pltpu.VMEM((B, SROWS, 128), jnp.float32),
            pltpu.VMEM((SQ, H_PER * DH), jnp.bfloat16),
            pltpu.VMEM((B, N_DEV, RC, D_MODEL), jnp.int8),
            pltpu.VMEM((B, N_DEV, SROWS, 128), jnp.float32),
            pltpu.VMEM((B, N_DEV, RC, D_MODEL), jnp.int8),
            pltpu.VMEM((B, N_DEV, SROWS, 128), jnp.float32),
            pltpu.VMEM((B, RC, D_MODEL), jnp.int8),
            pltpu.VMEM((B, SROWS, 128), jnp.float32),
            pltpu.SemaphoreType.DMA((2 * B * N_DEV,)),
            pltpu.SemaphoreType.DMA((2 * B * N_DEV,)),
            pltpu.SemaphoreType.DMA((2 * B * N_DEV,)),
            pltpu.SemaphoreType.DMA((2 * B * N_DEV,)),
        ],
        compiler_params=pltpu.CompilerParams(collective_id=0),
    )(xb, wq, k, v, wo)
    return out.reshape(B, SQ, D_MODEL)
